# Initial kernel scaffold; baseline (speedup 1.0000x reference)
import jax
import jax.numpy as jnp
from jax.experimental import pallas as pl


def kernel(x):
    raise NotImplementedError("write your pallas kernel here")


# speedup vs baseline: 2.6160x; 2.6160x over previous
import jax
import jax.numpy as jnp
import numpy as np
from jax.experimental import pallas as pl
from jax.experimental.pallas import tpu as pltpu

_TEMP = 0.8
_K = 50
_B = 128
_V = 100000
_VP = 102400
_W = 2048
_BR = 8
_NCH = _VP // _W
_NEG = np.float32(-1e30)
_INT_MIN = np.int32(-2147483648)


def _monotonic(bits):
    return jnp.where(bits < 0, bits ^ jnp.int32(0x7FFFFFFF), bits)


def _rotl(x, d):
    return jnp.left_shift(x, d) | jax.lax.shift_right_logical(x, 32 - d)


def _threefry_bits(p):
    ks0 = jnp.int32(0)
    ks1 = jnp.int32(1234)
    ks2 = jnp.int32(1234 ^ 0x1BD11BDA)
    x0 = jnp.zeros_like(p)
    x1 = p + ks1
    r0 = (13, 15, 26, 6)
    r1 = (17, 29, 16, 24)
    sched = ((r0, ks1, ks2, 1), (r1, ks2, ks0, 2), (r0, ks0, ks1, 3),
             (r1, ks1, ks2, 4), (r0, ks2, ks0, 5))
    for rs, a, b, i in sched:
        for r in rs:
            x0 = x0 + x1
            x1 = _rotl(x1, r)
            x1 = x0 ^ x1
        x0 = x0 + a
        x1 = x1 + b + jnp.int32(i)
    return x0 ^ x1


def _gumbel_from_bits(bits):
    fb = jax.lax.shift_right_logical(bits, 9) | jnp.int32(0x3F800000)
    u = jax.lax.bitcast_convert_type(fb, jnp.float32) - jnp.float32(1.0)
    tiny = jnp.float32(1.1754943508222875e-38)
    uu = jnp.maximum(tiny, u * (jnp.float32(1.0) - tiny) + tiny)
    return -jnp.log(-jnp.log(uu))


def _body(x_ref, probs_ref, nt_ref, y_ref):
    blk = pl.program_id(0)

    def p1(ch, m):
        sl = pl.ds(pl.multiple_of(ch * _W, _W), _W)
        l = x_ref[:, sl] / jnp.float32(_TEMP)
        y_ref[:, sl] = _monotonic(jax.lax.bitcast_convert_type(l, jnp.int32))
        return jnp.maximum(m, jnp.max(l, axis=1, keepdims=True))

    m = jax.lax.fori_loop(
        0, _NCH, p1, jnp.full((_BR, 1), -jnp.inf, jnp.float32))

    def pbit(t, pref):
        bit = 31 - t
        cand = pref | jnp.left_shift(jnp.int32(1), bit)
        s = cand ^ _INT_MIN

        def pcnt(ch, c):
            sl = pl.ds(pl.multiple_of(ch * _W, _W), _W)
            yv = y_ref[:, sl]
            return c + jnp.sum((yv >= s).astype(jnp.int32), axis=1,
                               keepdims=True)

        cnt = jax.lax.fori_loop(0, _NCH, pcnt,
                                jnp.zeros((_BR, 1), jnp.int32))
        return jnp.where(cnt >= _K, cand, pref)

    pref = jax.lax.fori_loop(0, 32, pbit, jnp.zeros((_BR, 1), jnp.int32))
    y_star = pref ^ _INT_MIN
    bits_star = jnp.where(y_star < 0, y_star ^ jnp.int32(0x7FFFFFFF), y_star)
    t_f = jax.lax.bitcast_convert_type(bits_star, jnp.float32)

    row = blk * _BR + jax.lax.broadcasted_iota(jnp.int32, (_BR, _W), 0)
    row_v = row * _V

    def p3(ch, carry):
        sacc, best, bidx = carry
        off = pl.multiple_of(ch * _W, _W)
        sl = pl.ds(off, _W)
        l = x_ref[:, sl] / jnp.float32(_TEMP)
        kept = l >= t_f
        e = jnp.where(kept, jnp.exp(l - m), jnp.float32(0.0))
        probs_ref[:, sl] = e
        sacc = sacc + jnp.sum(e, axis=1, keepdims=True)
        vg = ch * _W + jax.lax.broadcasted_iota(jnp.int32, (_BR, _W), 1)
        g = _gumbel_from_bits(_threefry_bits(row_v + vg))
        score = jnp.where(kept, l + g, _NEG)
        cb = jnp.max(score, axis=1, keepdims=True)
        ci = jnp.min(jnp.where(score == cb, vg, jnp.int32(2**30)),
                     axis=1, keepdims=True)
        upd = (cb > best) | ((cb == best) & (ci < bidx))
        best = jnp.where(upd, cb, best)
        bidx = jnp.where(upd, ci, bidx)
        return sacc, best, bidx

    sacc, best, bidx = jax.lax.fori_loop(
        0, _NCH, p3,
        (jnp.zeros((_BR, 1), jnp.float32),
         jnp.full((_BR, 1), -jnp.inf, jnp.float32),
         jnp.full((_BR, 1), 2**30, jnp.int32)))

    inv = jnp.float32(1.0) / sacc

    def p4(ch, _):
        sl = pl.ds(pl.multiple_of(ch * _W, _W), _W)
        probs_ref[:, sl] = probs_ref[:, sl] * inv
        return 0

    jax.lax.fori_loop(0, _NCH, p4, 0)
    nt_ref[...] = jnp.broadcast_to(bidx, (_BR, 128))


def kernel(x):
    xp = jnp.pad(x, ((0, 0), (0, _VP - _V)), constant_values=-jnp.inf)
    probs_p, nt = pl.pallas_call(
        _body,
        grid=(_B // _BR,),
        in_specs=[pl.BlockSpec((_BR, _VP), lambda i: (i, 0))],
        out_specs=[pl.BlockSpec((_BR, _VP), lambda i: (i, 0)),
                   pl.BlockSpec((_BR, 128), lambda i: (i, 0))],
        out_shape=[jax.ShapeDtypeStruct((_B, _VP), jnp.float32),
                   jax.ShapeDtypeStruct((_B, 128), jnp.int32)],
        scratch_shapes=[pltpu.VMEM((_BR, _VP), jnp.int32)],
    )(xp)
    return probs_p[:, :_V], nt[:, 0]


# speedup vs baseline: 8.0681x; 3.0841x over previous
import jax
import jax.numpy as jnp
import numpy as np
from jax.experimental import pallas as pl
from jax.experimental.pallas import tpu as pltpu

_TEMP = 0.8
_K = 50
_B = 128
_V = 100000
_VP = 102400
_W = 2048
_BR = 8
_NCH = _VP // _W
_NEG = np.float32(-1e30)
_INT_MIN = np.int32(-2147483648)


def _monotonic(bits):
    return jnp.where(bits < 0, bits ^ jnp.int32(0x7FFFFFFF), bits)


def _rotl(x, d):
    return jnp.left_shift(x, d) | jax.lax.shift_right_logical(x, 32 - d)


def _threefry_bits(p):
    ks0 = jnp.int32(0)
    ks1 = jnp.int32(1234)
    ks2 = jnp.int32(1234 ^ 0x1BD11BDA)
    x0 = jnp.zeros_like(p)
    x1 = p + ks1
    r0 = (13, 15, 26, 6)
    r1 = (17, 29, 16, 24)
    sched = ((r0, ks1, ks2, 1), (r1, ks2, ks0, 2), (r0, ks0, ks1, 3),
             (r1, ks1, ks2, 4), (r0, ks2, ks0, 5))
    for rs, a, b, i in sched:
        for r in rs:
            x0 = x0 + x1
            x1 = _rotl(x1, r)
            x1 = x0 ^ x1
        x0 = x0 + a
        x1 = x1 + b + jnp.int32(i)
    return x0 ^ x1


def _gumbel_from_bits(bits):
    fb = jax.lax.shift_right_logical(bits, 9) | jnp.int32(0x3F800000)
    u = jax.lax.bitcast_convert_type(fb, jnp.float32) - jnp.float32(1.0)
    tiny = jnp.float32(1.1754943508222875e-38)
    uu = jnp.maximum(tiny, u * (jnp.float32(1.0) - tiny) + tiny)
    return -jnp.log(-jnp.log(uu))


def _body(x_ref, probs_ref, nt_ref, yh_ref, yl_ref):
    blk = pl.program_id(0)

    def p1(ch, m):
        sl = pl.ds(pl.multiple_of(ch * _W, _W), _W)
        l = x_ref[:, sl] / jnp.float32(_TEMP)
        u = _monotonic(jax.lax.bitcast_convert_type(l, jnp.int32)) ^ _INT_MIN
        h = jax.lax.shift_right_logical(u, 16)
        lo = u & jnp.int32(0xFFFF)
        yh_ref[:, sl] = (h - jnp.int32(32768)).astype(jnp.int16)
        yl_ref[:, sl] = (lo - jnp.int32(32768)).astype(jnp.int16)
        return jnp.maximum(m, jnp.max(l, axis=1, keepdims=True))

    m = jax.lax.fori_loop(
        0, _NCH, p1, jnp.full((_BR, 1), -jnp.inf, jnp.float32))

    one16 = np.int16(1)
    zero16 = np.int16(0)

    def _count_hi(s_h16):
        def pcnt(ch, acc):
            sl = pl.ds(pl.multiple_of(ch * _W, _W), _W)
            yv = yh_ref[:, sl]
            return acc + jnp.where(yv >= s_h16, one16, zero16)

        acc = jax.lax.fori_loop(0, _NCH, pcnt,
                                jnp.zeros((_BR, _W), jnp.int16))
        return jnp.sum(acc.astype(jnp.int32), axis=1, keepdims=True)

    def pbit_hi(t, pref):
        bit = 15 - t
        cand = pref | jnp.left_shift(jnp.int32(1), bit)
        cnt = _count_hi((cand - jnp.int32(32768)).astype(jnp.int16))
        return jnp.where(cnt >= _K, cand, pref)

    pref_h = jax.lax.fori_loop(0, 16, pbit_hi,
                               jnp.zeros((_BR, 1), jnp.int32))
    ph16 = (pref_h - jnp.int32(32768)).astype(jnp.int16)

    cnt_gt = _count_hi((pref_h + 1 - jnp.int32(32768)).astype(jnp.int16))
    cnt_gt = jnp.where(pref_h >= jnp.int32(65535),
                       jnp.zeros_like(cnt_gt), cnt_gt)

    def pbit_lo(t, pref):
        bit = 15 - t
        cand = pref | jnp.left_shift(jnp.int32(1), bit)
        s_l16 = (cand - jnp.int32(32768)).astype(jnp.int16)

        def pcnt(ch, acc):
            sl = pl.ds(pl.multiple_of(ch * _W, _W), _W)
            hit = (yh_ref[:, sl] == ph16) & (yl_ref[:, sl] >= s_l16)
            return acc + jnp.where(hit, one16, zero16)

        acc = jax.lax.fori_loop(0, _NCH, pcnt,
                                jnp.zeros((_BR, _W), jnp.int16))
        cnt = cnt_gt + jnp.sum(acc.astype(jnp.int32), axis=1, keepdims=True)
        return jnp.where(cnt >= _K, cand, pref)

    pref_l = jax.lax.fori_loop(0, 16, pbit_lo,
                               jnp.zeros((_BR, 1), jnp.int32))

    u_star = jnp.left_shift(pref_h, 16) | pref_l
    y_star = u_star ^ _INT_MIN
    bits_star = jnp.where(y_star < 0, y_star ^ jnp.int32(0x7FFFFFFF), y_star)
    t_f = jax.lax.bitcast_convert_type(bits_star, jnp.float32)

    row = blk * _BR + jax.lax.broadcasted_iota(jnp.int32, (_BR, _W), 0)
    row_v = row * _V

    def p3(ch, carry):
        sacc, best, bidx = carry
        off = pl.multiple_of(ch * _W, _W)
        sl = pl.ds(off, _W)
        l = x_ref[:, sl] / jnp.float32(_TEMP)
        kept = l >= t_f
        e = jnp.where(kept, jnp.exp(l - m), jnp.float32(0.0))
        probs_ref[:, sl] = e
        sacc = sacc + jnp.sum(e, axis=1, keepdims=True)
        vg = ch * _W + jax.lax.broadcasted_iota(jnp.int32, (_BR, _W), 1)
        g = _gumbel_from_bits(_threefry_bits(row_v + vg))
        score = jnp.where(kept, l + g, _NEG)
        cb = jnp.max(score, axis=1, keepdims=True)
        ci = jnp.min(jnp.where(score == cb, vg, jnp.int32(2**30)),
                     axis=1, keepdims=True)
        upd = (cb > best) | ((cb == best) & (ci < bidx))
        best = jnp.where(upd, cb, best)
        bidx = jnp.where(upd, ci, bidx)
        return sacc, best, bidx

    sacc, best, bidx = jax.lax.fori_loop(
        0, _NCH, p3,
        (jnp.zeros((_BR, 1), jnp.float32),
         jnp.full((_BR, 1), -jnp.inf, jnp.float32),
         jnp.full((_BR, 1), 2**30, jnp.int32)))

    inv = jnp.float32(1.0) / sacc

    def p4(ch, _):
        sl = pl.ds(pl.multiple_of(ch * _W, _W), _W)
        probs_ref[:, sl] = probs_ref[:, sl] * inv
        return 0

    jax.lax.fori_loop(0, _NCH, p4, 0)
    nt_ref[...] = jnp.broadcast_to(bidx, (_BR, 128))


def kernel(x):
    xp = jnp.pad(x, ((0, 0), (0, _VP - _V)), constant_values=-jnp.inf)
    probs_p, nt = pl.pallas_call(
        _body,
        grid=(_B // _BR,),
        in_specs=[pl.BlockSpec((_BR, _VP), lambda i: (i, 0))],
        out_specs=[pl.BlockSpec((_BR, _VP), lambda i: (i, 0)),
                   pl.BlockSpec((_BR, 128), lambda i: (i, 0))],
        out_shape=[jax.ShapeDtypeStruct((_B, _VP), jnp.float32),
                   jax.ShapeDtypeStruct((_B, 128), jnp.int32)],
        scratch_shapes=[pltpu.VMEM((_BR, _VP), jnp.int16),
                        pltpu.VMEM((_BR, _VP), jnp.int16)],
    )(xp)
    return probs_p[:, :_V], nt[:, 0]
